# Initial kernel scaffold; baseline (speedup 1.0000x reference)
import jax
import jax.numpy as jnp
from jax.experimental import pallas as pl


def kernel(inputs, rel_embeddings):
    raise NotImplementedError("write your pallas kernel here")


# speedup vs baseline: 27.2154x; 27.2154x over previous
import jax
import jax.numpy as jnp
from jax.experimental import pallas as pl


def _build_phase_table(rel_embeddings):
    flat = jnp.flip(rel_embeddings, axis=0).reshape(-1)
    flat = jnp.concatenate([flat, jnp.zeros((128,), flat.dtype)])
    rows = [
        jax.lax.dynamic_slice(flat, (16 * (7 - r),), (65536,)).reshape(512, 128)
        for r in range(8)
    ]
    return jnp.stack(rows)


def _copy_block(q_ref, o_ref):
    b = pl.program_id(0)
    o_ref[...] = q_ref[:, pl.ds(255 - b, 256), :]


def kernel(inputs, rel_embeddings):
    del inputs
    q = _build_phase_table(rel_embeddings)
    out = pl.pallas_call(
        _copy_block,
        grid=(256,),
        in_specs=[pl.BlockSpec((8, 512, 128), lambda i: (0, 0, 0))],
        out_specs=pl.BlockSpec((8, 256, 128), lambda i: (i, 0, 0)),
        out_shape=jax.ShapeDtypeStruct((2048, 256, 128), jnp.float32),
    )(q)
    return out.reshape(2048, 2048, 16)


# speedup vs baseline: 30.0195x; 1.1030x over previous
import jax
import jax.numpy as jnp
from jax.experimental import pallas as pl
from jax.experimental.pallas import tpu as pltpu

_NBLK = 256
_DEPTH_INFLIGHT = 8


def _build_phase_table(rel_embeddings):
    flat = jnp.flip(rel_embeddings, axis=0).reshape(-1)
    flat = jnp.concatenate([flat, jnp.zeros((128,), flat.dtype)])
    rows = [
        jax.lax.dynamic_slice(flat, (16 * (7 - r),), (65536,)).reshape(512, 128)
        for r in range(8)
    ]
    return jnp.stack(rows)


def _dma_kernel(q_hbm, out_hbm, q_vmem, load_sem, sems):
    load = pltpu.make_async_copy(q_hbm, q_vmem, load_sem)
    load.start()
    load.wait()

    def _block_copy(b):
        return pltpu.make_async_copy(
            q_vmem.at[:, pl.ds(255 - b, 256), :],
            out_hbm.at[pl.ds(8 * b, 8), :, :],
            sems.at[jax.lax.rem(b, _DEPTH_INFLIGHT)],
        )

    def body(b, _):
        _block_copy(b).start()

        @pl.when(b >= _DEPTH_INFLIGHT - 1)
        def _():
            _block_copy(b - (_DEPTH_INFLIGHT - 1)).wait()

        return 0

    jax.lax.fori_loop(0, _NBLK, body, 0)

    def tail(b, _):
        _block_copy(b).wait()
        return 0

    jax.lax.fori_loop(_NBLK - (_DEPTH_INFLIGHT - 1), _NBLK, tail, 0)


def kernel(inputs, rel_embeddings):
    del inputs
    q = _build_phase_table(rel_embeddings)
    out = pl.pallas_call(
        _dma_kernel,
        in_specs=[pl.BlockSpec(memory_space=pl.ANY)],
        out_specs=pl.BlockSpec(memory_space=pl.ANY),
        out_shape=jax.ShapeDtypeStruct((2048, 256, 128), jnp.float32),
        scratch_shapes=[
            pltpu.VMEM((8, 512, 128), jnp.float32),
            pltpu.SemaphoreType.DMA,
            pltpu.SemaphoreType.DMA((_DEPTH_INFLIGHT,)),
        ],
    )(q)
    return out.reshape(2048, 2048, 16)
